# baseline (speedup 1.0000x reference)
import functools

import jax
import jax.numpy as jnp
from jax.experimental import pallas as pl
from jax.experimental.pallas import tpu as pltpu

_EPAD = 128


def _router_kernel(x_ref, lng_ref, lnb_ref, W1_ref, b1_ref, W2_ref, b2_ref,
                   idx_ref, probs_ref, aux_ref,
                   acc_ref, w1s_ref, xn_ref, h_ref,
                   *, n_tokens, n_tok_blocks, n_col, n_experts):
    t = pl.program_id(0)
    c = pl.program_id(1)

    @pl.when((t == 0) & (c == 0))
    def _init():
        acc_ref[...] = jnp.zeros_like(acc_ref)

    @pl.when(c == 0)
    def _layernorm():
        xb = x_ref[...]
        mu = jnp.mean(xb, axis=1, keepdims=True)
        xc = xb - mu
        var = jnp.mean(xc * xc, axis=1, keepdims=True)
        xn_ref[...] = xc / jnp.sqrt(var + 1e-5) * lng_ref[...] + lnb_ref[...]

    @pl.when(t == 0)
    def _stash():
        w1s_ref[c] = W1_ref[...]

    w1c = w1s_ref[c]
    hc = jnp.dot(xn_ref[...], w1c, preferred_element_type=jnp.float32) + b1_ref[0]
    h_ref[c] = jnp.maximum(hc, 0.0)

    @pl.when(c == n_col - 1)
    def _router_tail():
        h = jnp.concatenate([h_ref[k] for k in range(n_col)], axis=1)
        logits = jnp.dot(h, W2_ref[...], preferred_element_type=jnp.float32) \
            + b2_ref[...]

        m = jnp.max(logits, axis=1, keepdims=True)
        e = jnp.exp(logits - m)
        denom = jnp.sum(e, axis=1, keepdims=True)
        probs = e / denom

        acc_ref[...] += jnp.sum(probs, axis=0, keepdims=True)

        iota = jax.lax.broadcasted_iota(jnp.int32, probs.shape, 1)
        big = jnp.int32(2 ** 30)
        p1 = jnp.max(probs, axis=1, keepdims=True)
        i1 = jnp.min(jnp.where(probs == p1, iota, big), axis=1, keepdims=True)
        pm = jnp.where(iota == i1, -1.0, probs)
        p2 = jnp.max(pm, axis=1, keepdims=True)
        i2 = jnp.min(jnp.where(pm == p2, iota, big), axis=1, keepdims=True)
        s = p1 + p2

        idx_ref[...] = jnp.concatenate([i1, i2], axis=1)
        probs_ref[...] = jnp.concatenate([p1 / s, p2 / s], axis=1)

        @pl.when(t == n_tok_blocks - 1)
        def _finalize():
            rp = acc_ref[...] / jnp.float32(n_tokens)
            aux = jnp.sum(rp * jnp.log(rp * jnp.float32(n_experts) + 1e-9),
                          axis=1, keepdims=True)
            aux_ref[...] = aux


def kernel(x, ln_g, ln_b, W1, b1, W2, b2):
    B, S, H = x.shape
    E = W2.shape[1]
    N = B * S
    T = min(512, N)
    n_tok_blocks = N // T
    C = min(256, H)
    n_col = H // C

    xf = x.reshape(N, H)
    lng = ln_g.reshape(1, H)
    lnb = ln_b.reshape(1, H)
    b1r = b1.reshape(n_col, 1, C)
    W2p = jnp.zeros((H, _EPAD), W2.dtype).at[:, :E].set(W2)
    b2p = jnp.full((1, _EPAD), -1e30, b2.dtype).at[0, :E].set(b2)

    grid = (n_tok_blocks, n_col)
    kern = functools.partial(_router_kernel, n_tokens=N,
                             n_tok_blocks=n_tok_blocks, n_col=n_col,
                             n_experts=E)
    last = n_col - 1
    idx, probs, aux = pl.pallas_call(
        kern,
        grid=grid,
        in_specs=[
            pl.BlockSpec((T, H), lambda t, c: (t, 0)),
            pl.BlockSpec((1, H), lambda t, c: (0, 0)),
            pl.BlockSpec((1, H), lambda t, c: (0, 0)),
            pl.BlockSpec((H, C),
                         lambda t, c: (0, jnp.where(t == 0, c, last))),
            pl.BlockSpec((1, 1, C), lambda t, c: (c, 0, 0)),
            pl.BlockSpec((H, _EPAD), lambda t, c: (0, 0)),
            pl.BlockSpec((1, _EPAD), lambda t, c: (0, 0)),
        ],
        out_specs=[
            pl.BlockSpec((T, 2), lambda t, c: (t, 0)),
            pl.BlockSpec((T, 2), lambda t, c: (t, 0)),
            pl.BlockSpec((1, 1), lambda t, c: (0, 0)),
        ],
        out_shape=[
            jax.ShapeDtypeStruct((N, 2), jnp.int32),
            jax.ShapeDtypeStruct((N, 2), jnp.float32),
            jax.ShapeDtypeStruct((1, 1), jnp.float32),
        ],
        scratch_shapes=[
            pltpu.VMEM((1, _EPAD), jnp.float32),
            pltpu.VMEM((n_col, H, C), jnp.float32),
            pltpu.VMEM((T, H), jnp.float32),
            pltpu.VMEM((n_col, T, C), jnp.float32),
        ],
        compiler_params=pltpu.CompilerParams(
            dimension_semantics=("arbitrary", "arbitrary"),
        ),
    )(xf, lng, lnb, W1, b1r, W2p, b2p)

    top_k_indices = idx.reshape(B, S, 2)
    top_k_probs = probs.reshape(B, S, 2)
    aux_loss = aux[0, 0]
    return (top_k_indices, top_k_probs, aux_loss)


# speedup vs baseline: 1.3400x; 1.3400x over previous
import functools

import jax
import jax.numpy as jnp
from jax.experimental import pallas as pl
from jax.experimental.pallas import tpu as pltpu


def _pipeline_step(s, x_ref, lng_ref, lnb_ref, W1_ref, b1_ref, W2_ref, b2_ref,
                   idx_ref, probs_ref, acc_ref,
                   xn_w, xn_r, lg_w, lg_r, n_blocks):
    xb = x_ref[...]
    mu = jnp.mean(xb, axis=1, keepdims=True)
    xc = xb - mu
    var = jnp.mean(xc * xc, axis=1, keepdims=True)
    xn_w[...] = xc / jnp.sqrt(var + 1e-5) * lng_ref[...] + lnb_ref[...]

    h = jnp.dot(xn_r[...], W1_ref[...],
                preferred_element_type=jnp.float32) + b1_ref[...]
    h = jnp.maximum(h, 0.0)
    lg_w[...] = jnp.dot(h, W2_ref[...],
                        preferred_element_type=jnp.float32) + b2_ref[...]

    logits = lg_r[...]
    m = jnp.max(logits, axis=1, keepdims=True)
    e = jnp.exp(logits - m)
    denom = jnp.sum(e, axis=1, keepdims=True)
    probs = e / denom

    valid = (s >= 2) & (s <= n_blocks + 1)
    colsum = jnp.sum(probs, axis=0, keepdims=True)
    acc_ref[...] += jnp.where(valid, colsum, 0.0)

    iota = jax.lax.broadcasted_iota(jnp.int32, probs.shape, 1)
    big = jnp.int32(2 ** 30)
    p1 = jnp.max(probs, axis=1, keepdims=True)
    i1 = jnp.min(jnp.where(probs == p1, iota, big), axis=1, keepdims=True)
    pm = jnp.where(iota == i1, -1.0, probs)
    p2 = jnp.max(pm, axis=1, keepdims=True)
    i2 = jnp.min(jnp.where(pm == p2, iota, big), axis=1, keepdims=True)
    ssum = p1 + p2

    idx_ref[...] = jnp.concatenate([i1, i2], axis=1)
    probs_ref[...] = jnp.concatenate([p1 / ssum, p2 / ssum], axis=1)


def _router_kernel(x_ref, lng_ref, lnb_ref, W1_ref, b1_ref, W2_ref, b2_ref,
                   idx_ref, probs_ref, aux_ref,
                   acc_ref, xn_a, xn_b, lg_a, lg_b,
                   *, n_tokens, n_blocks, n_experts):
    s = pl.program_id(0)

    @pl.when(s == 0)
    def _init():
        acc_ref[...] = jnp.zeros_like(acc_ref)

    common = (x_ref, lng_ref, lnb_ref, W1_ref, b1_ref, W2_ref, b2_ref,
              idx_ref, probs_ref, acc_ref)

    @pl.when(s % 2 == 0)
    def _even():
        _pipeline_step(s, *common, xn_a, xn_b, lg_b, lg_a, n_blocks)

    @pl.when(s % 2 == 1)
    def _odd():
        _pipeline_step(s, *common, xn_b, xn_a, lg_a, lg_b, n_blocks)

    @pl.when(s == n_blocks + 1)
    def _finalize():
        rp = acc_ref[...] / jnp.float32(n_tokens)
        aux = jnp.sum(rp * jnp.log(rp * jnp.float32(n_experts) + 1e-9),
                      axis=1, keepdims=True)
        aux_ref[...] = aux


def kernel(x, ln_g, ln_b, W1, b1, W2, b2):
    B, S, H = x.shape
    E = W2.shape[1]
    N = B * S
    T = min(512, N)
    n_blocks = N // T

    xf = x.reshape(N, H)
    lng = ln_g.reshape(1, H)
    lnb = ln_b.reshape(1, H)
    b1r = b1.reshape(1, H)
    b2r = b2.reshape(1, E)

    last = n_blocks - 1
    grid = (n_blocks + 2,)
    kern = functools.partial(_router_kernel, n_tokens=N, n_blocks=n_blocks,
                             n_experts=E)
    idx, probs, aux = pl.pallas_call(
        kern,
        grid=grid,
        in_specs=[
            pl.BlockSpec((T, H), lambda s: (jnp.minimum(s, last), 0)),
            pl.BlockSpec((1, H), lambda s: (0, 0)),
            pl.BlockSpec((1, H), lambda s: (0, 0)),
            pl.BlockSpec((H, H), lambda s: (0, 0)),
            pl.BlockSpec((1, H), lambda s: (0, 0)),
            pl.BlockSpec((H, E), lambda s: (0, 0)),
            pl.BlockSpec((1, E), lambda s: (0, 0)),
        ],
        out_specs=[
            pl.BlockSpec((T, 2),
                         lambda s: (jnp.clip(s - 2, 0, last), 0)),
            pl.BlockSpec((T, 2),
                         lambda s: (jnp.clip(s - 2, 0, last), 0)),
            pl.BlockSpec((1, 1), lambda s: (0, 0)),
        ],
        out_shape=[
            jax.ShapeDtypeStruct((N, 2), jnp.int32),
            jax.ShapeDtypeStruct((N, 2), jnp.float32),
            jax.ShapeDtypeStruct((1, 1), jnp.float32),
        ],
        scratch_shapes=[
            pltpu.VMEM((1, E), jnp.float32),
            pltpu.VMEM((T, H), jnp.float32),
            pltpu.VMEM((T, H), jnp.float32),
            pltpu.VMEM((T, E), jnp.float32),
            pltpu.VMEM((T, E), jnp.float32),
        ],
        compiler_params=pltpu.CompilerParams(
            dimension_semantics=("arbitrary",),
        ),
    )(xf, lng, lnb, W1, b1r, W2, b2r)

    top_k_indices = idx.reshape(B, S, 2)
    top_k_probs = probs.reshape(B, S, 2)
    aux_loss = aux[0, 0]
    return (top_k_indices, top_k_probs, aux_loss)


# speedup vs baseline: 1.5688x; 1.1707x over previous
import functools

import jax
import jax.numpy as jnp
from jax.experimental import pallas as pl
from jax.experimental.pallas import tpu as pltpu


def _mlp_stage(x_ref, lng_ref, lnb_ref, W1_ref, b1_ref, W2_ref, b2_ref, lg_w):
    xb = x_ref[...]
    mu = jnp.mean(xb, axis=1, keepdims=True)
    xc = xb - mu
    var = jnp.mean(xc * xc, axis=1, keepdims=True)
    xn = xc / jnp.sqrt(var + 1e-5) * lng_ref[...] + lnb_ref[...]
    h = jnp.dot(xn, W1_ref[...], preferred_element_type=jnp.float32) + b1_ref[...]
    h = jnp.maximum(h, 0.0)
    lg_w[...] = jnp.dot(h, W2_ref[...],
                        preferred_element_type=jnp.float32) + b2_ref[...]


def _tail_stage(lg_r, idx_ref, probs_ref, acc_ref, valid):
    logits = lg_r[...]
    m = jnp.max(logits, axis=1, keepdims=True)
    e = jnp.exp(logits - m)
    denom = jnp.sum(e, axis=1, keepdims=True)
    probs = e / denom

    colsum = jnp.sum(probs, axis=0, keepdims=True)
    acc_ref[...] += jnp.where(valid, colsum, 0.0)

    iota = jax.lax.broadcasted_iota(jnp.int32, probs.shape, 1)
    big = jnp.int32(2 ** 30)
    p1 = jnp.max(probs, axis=1, keepdims=True)
    i1 = jnp.min(jnp.where(probs == p1, iota, big), axis=1, keepdims=True)
    pm = jnp.where(iota == i1, -1.0, probs)
    p2 = jnp.max(pm, axis=1, keepdims=True)
    i2 = jnp.min(jnp.where(pm == p2, iota, big), axis=1, keepdims=True)
    ssum = p1 + p2

    idx_ref[...] = jnp.concatenate([i1, i2], axis=1)
    probs_ref[...] = jnp.concatenate([p1 / ssum, p2 / ssum], axis=1)


def _router_kernel(x_ref, lng_ref, lnb_ref, W1_ref, b1_ref, W2_ref, b2_ref,
                   idx_ref, probs_ref, aux_ref,
                   acc_ref, lg_a, lg_b,
                   *, n_tokens, n_blocks, n_experts):
    s = pl.program_id(0)

    @pl.when(s == 0)
    def _init():
        acc_ref[...] = jnp.zeros_like(acc_ref)

    mlp_args = (x_ref, lng_ref, lnb_ref, W1_ref, b1_ref, W2_ref, b2_ref)

    valid = s >= 1

    @pl.when((s % 2 == 0) & (s < n_blocks))
    def _even():
        _mlp_stage(*mlp_args, lg_a)
        _tail_stage(lg_b, idx_ref, probs_ref, acc_ref, valid)

    @pl.when((s % 2 == 1) & (s < n_blocks))
    def _odd():
        _mlp_stage(*mlp_args, lg_b)
        _tail_stage(lg_a, idx_ref, probs_ref, acc_ref, valid)

    @pl.when(s == n_blocks)
    def _last_tail():
        last_parity_a = (n_blocks - 1) % 2 == 0
        _tail_stage(lg_a if last_parity_a else lg_b,
                    idx_ref, probs_ref, acc_ref, True)
        rp = acc_ref[...] / jnp.float32(n_tokens)
        aux = jnp.sum(rp * jnp.log(rp * jnp.float32(n_experts) + 1e-9),
                      axis=1, keepdims=True)
        aux_ref[...] = aux


def kernel(x, ln_g, ln_b, W1, b1, W2, b2):
    B, S, H = x.shape
    E = W2.shape[1]
    N = B * S
    T = min(1024, N)
    n_blocks = N // T

    xf = x.reshape(N, H)
    lng = ln_g.reshape(1, H)
    lnb = ln_b.reshape(1, H)
    b1r = b1.reshape(1, H)
    b2r = b2.reshape(1, E)

    last = n_blocks - 1
    grid = (n_blocks + 1,)
    kern = functools.partial(_router_kernel, n_tokens=N, n_blocks=n_blocks,
                             n_experts=E)
    idx, probs, aux = pl.pallas_call(
        kern,
        grid=grid,
        in_specs=[
            pl.BlockSpec((T, H), lambda s: (jnp.minimum(s, last), 0)),
            pl.BlockSpec((1, H), lambda s: (0, 0)),
            pl.BlockSpec((1, H), lambda s: (0, 0)),
            pl.BlockSpec((H, H), lambda s: (0, 0)),
            pl.BlockSpec((1, H), lambda s: (0, 0)),
            pl.BlockSpec((H, E), lambda s: (0, 0)),
            pl.BlockSpec((1, E), lambda s: (0, 0)),
        ],
        out_specs=[
            pl.BlockSpec((T, 2),
                         lambda s: (jnp.clip(s - 1, 0, last), 0)),
            pl.BlockSpec((T, 2),
                         lambda s: (jnp.clip(s - 1, 0, last), 0)),
            pl.BlockSpec((1, 1), lambda s: (0, 0)),
        ],
        out_shape=[
            jax.ShapeDtypeStruct((N, 2), jnp.int32),
            jax.ShapeDtypeStruct((N, 2), jnp.float32),
            jax.ShapeDtypeStruct((1, 1), jnp.float32),
        ],
        scratch_shapes=[
            pltpu.VMEM((1, E), jnp.float32),
            pltpu.VMEM((T, E), jnp.float32),
            pltpu.VMEM((T, E), jnp.float32),
        ],
        compiler_params=pltpu.CompilerParams(
            dimension_semantics=("arbitrary",),
        ),
    )(xf, lng, lnb, W1, b1r, W2, b2r)

    top_k_indices = idx.reshape(B, S, 2)
    top_k_probs = probs.reshape(B, S, 2)
    aux_loss = aux[0, 0]
    return (top_k_indices, top_k_probs, aux_loss)
